# baseline (speedup 1.0000x reference)
import dataclasses
import functools

import jax
import jax.numpy as jnp
from jax import lax
from jax.experimental import pallas as pl
from jax.experimental.pallas import tpu as pltpu
from jax.experimental.pallas import tpu_sc as plsc

_NC = 2
_NS = 16
_NW = _NC * _NS
_LANES = 16
_C = 64


def _project_table(values_bf, w9_bf, m_block):
    n_coarse, d = values_bf.shape
    fe, _, nf = w9_bf.shape

    def body(v_ref, w_ref, o_ref):
        o_ref[0] = lax.dot_general(
            v_ref[...], w_ref[0], (((1,), (0,)), ((), ())),
            preferred_element_type=jnp.float32).astype(jnp.bfloat16)

    return pl.pallas_call(
        body,
        grid=(n_coarse // m_block, fe),
        in_specs=[
            pl.BlockSpec((m_block, d), lambda m, k: (m, 0)),
            pl.BlockSpec((1, d, nf), lambda m, k: (k, 0, 0)),
        ],
        out_specs=pl.BlockSpec((1, m_block, nf), lambda m, k: (k, m, 0)),
        out_shape=jax.ShapeDtypeStruct((fe, n_coarse, nf), jnp.bfloat16),
    )(values_bf, w9_bf)


def _sc_compiler_params():
    cp = pltpu.CompilerParams(use_tc_tiling_on_sc=False)
    if "needs_layout_passes" in pltpu.CompilerParams.__dataclass_fields__:
        cp = dataclasses.replace(cp, needs_layout_passes=False)
    return cp


def _gather_sum(table, idx_pad, fe, nf, n_coarse, bpw, n_pad):
    n_ch = bpw // _C
    assert n_ch % 2 == 0
    rows_a = (n_coarse // _NS) & ~7
    rows_b = n_coarse - _NS * rows_a
    mesh = plsc.VectorSubcoreMesh(core_axis_name="c", subcore_axis_name="s")

    @functools.partial(
        pl.kernel,
        out_type=jax.ShapeDtypeStruct((n_pad * nf,), jnp.bfloat16),
        mesh=mesh,
        scratch_types=[
            pltpu.VMEM_SHARED((n_coarse, nf), jnp.bfloat16),
            pltpu.VMEM_SHARED((n_coarse, nf), jnp.bfloat16),
            pltpu.VMEM((bpw * nf,), jnp.bfloat16),
            pltpu.VMEM((_C, nf), jnp.bfloat16),
            pltpu.VMEM((_C, nf), jnp.bfloat16),
            pltpu.VMEM((bpw,), jnp.int32),
            pltpu.VMEM((bpw,), jnp.int32),
            pltpu.SemaphoreType.DMA,
            pltpu.SemaphoreType.DMA,
            pltpu.SemaphoreType.DMA,
            pltpu.SemaphoreType.DMA,
            pltpu.SemaphoreType.DMA,
            pltpu.SemaphoreType.DMA,
        ],
        compiler_params=_sc_compiler_params(),
    )
    def body(table_hbm, idx_hbm, out_hbm,
             sh0, sh1, acc_v, r0, r1, ix0, ix1,
             ss0, ss1, sg0, sg1, si0, si1):
        cid = lax.axis_index("c")
        sid = lax.axis_index("s")
        wid = sid * _NC + cid
        sh = (sh0, sh1)
        rows = (r0, r1)
        ix = (ix0, ix1)
        ss = (ss0, ss1)
        sg = (sg0, sg1)
        si = (si0, si1)

        def stage_copies(k, q):
            a = sid * rows_a
            yield (table_hbm.at[k, pl.ds(a, rows_a)],
                   sh[q].at[pl.ds(a, rows_a)], ss[q])
            b = _NS * rows_a
            yield (table_hbm.at[k, pl.ds(b, rows_b)],
                   sh[q].at[pl.ds(b, rows_b)], ss[q])

        def stage_start(k, q):
            cps = list(stage_copies(k, q))
            pltpu.async_copy(*cps[0])

            @pl.when(sid == 0)
            def _():
                pltpu.async_copy(*cps[1])

        def stage_wait(k, q):
            cps = list(stage_copies(k, q))
            pltpu.make_async_copy(*cps[0]).wait()

            @pl.when(sid == 0)
            def _():
                pltpu.make_async_copy(*cps[1]).wait()

        def idx_start(k, kp):
            pltpu.async_copy(idx_hbm.at[k, pl.ds(wid * bpw, bpw)], ix[kp],
                             si[kp])

        def idx_wait(k, kp):
            pltpu.make_async_copy(idx_hbm.at[k, pl.ds(wid * bpw, bpw)],
                                  ix[kp], si[kp]).wait()

        def gather(k, q, c, p):
            src = sh[q].at[ix[k % 2].at[pl.ds(c * _C, _C)]]
            pltpu.async_copy(src, rows[p], sg[p])

        def gather_wait(k, q, p):
            src = sh[q].at[ix[k % 2].at[pl.ds(0, _C)]]
            pltpu.make_async_copy(src, rows[p], sg[p]).wait()

        def accumulate(k, c, p):
            @pl.loop(0, _C, step=4)
            def _row(r):
                for rr in range(4):
                    for j in range(nf // (2 * _LANES)):
                        w = 2 * _LANES
                        s = pl.ds((c * _C + r + rr) * nf + j * w, w)
                        v = rows[p][r + rr, pl.ds(j * w, w)]
                        if k == 0:
                            acc_v[s] = v
                        else:
                            acc_v[s] = acc_v[s] + v

        stage_start(0, 0)
        stage_start(1, 1)
        idx_start(0, 0)
        idx_start(1, 1)
        idx_wait(0, 0)
        stage_wait(0, 0)
        plsc.subcore_barrier()

        for k in range(fe):
            q = k % 2
            gather(k, q, 0, 0)

            @pl.loop(0, n_ch, step=2)
            def _pair(c, k=k, q=q):
                gather_wait(k, q, 0)
                gather(k, q, c + 1, 1)
                accumulate(k, c, 0)
                gather_wait(k, q, 1)

                @pl.when(c + 2 < n_ch)
                def _():
                    gather(k, q, c + 2, 0)

                accumulate(k, c + 1, 1)

            if k + 2 <= fe - 1:
                idx_start(k + 2, q)
            if k + 1 <= fe - 1:
                idx_wait(k + 1, 1 - q)
                stage_wait(k + 1, 1 - q)
                plsc.subcore_barrier()
                if k + 2 <= fe - 1:
                    stage_start(k + 2, q)

        pltpu.sync_copy(acc_v, out_hbm.at[pl.ds(wid * bpw * nf, bpw * nf)])

    return body(table, idx_pad)


def kernel(lattice_coarse_values, neighbor_indices, weight):
    n_coarse, d = lattice_coarse_values.shape
    n_fine, fe = neighbor_indices.shape
    nf = weight.shape[1]

    vb = lattice_coarse_values.astype(jnp.bfloat16)
    w9b = weight.reshape(fe, d, nf).astype(jnp.bfloat16)
    table = _project_table(vb, w9b, m_block=2000)

    per_round = _NW * _C
    n_chunks = -(-n_fine // per_round)
    n_chunks += n_chunks % 2
    n_pad = n_chunks * per_round
    bpw = n_chunks * _C
    idx_pad = jnp.pad(neighbor_indices.T.astype(jnp.int32),
                      ((0, 0), (0, n_pad - n_fine)))

    out = _gather_sum(table, idx_pad, fe, nf, n_coarse, bpw, n_pad)
    return out[:n_fine * nf].astype(jnp.float32).reshape(n_fine, nf)


# speedup vs baseline: 1.4367x; 1.4367x over previous
import dataclasses
import functools

import jax
import jax.numpy as jnp
from jax import lax
from jax.experimental import pallas as pl
from jax.experimental.pallas import tpu as pltpu
from jax.experimental.pallas import tpu_sc as plsc

_NC = 2
_NS = 16
_NW = _NC * _NS
_LANES = 16
_C = 64


def _project_table(values_bf, w9_bf, m_block):
    n_coarse, d = values_bf.shape
    fe, _, nf = w9_bf.shape

    def body(v_ref, w_ref, o_ref):
        o_ref[0] = lax.dot_general(
            v_ref[...], w_ref[0], (((1,), (0,)), ((), ())),
            preferred_element_type=jnp.float32).astype(jnp.bfloat16)

    return pl.pallas_call(
        body,
        grid=(n_coarse // m_block, fe),
        in_specs=[
            pl.BlockSpec((m_block, d), lambda m, k: (m, 0)),
            pl.BlockSpec((1, d, nf), lambda m, k: (k, 0, 0)),
        ],
        out_specs=pl.BlockSpec((1, m_block, nf), lambda m, k: (k, m, 0)),
        out_shape=jax.ShapeDtypeStruct((fe, n_coarse, nf), jnp.bfloat16),
    )(values_bf, w9_bf)


def _sc_compiler_params():
    cp = pltpu.CompilerParams(use_tc_tiling_on_sc=False)
    if "needs_layout_passes" in pltpu.CompilerParams.__dataclass_fields__:
        cp = dataclasses.replace(cp, needs_layout_passes=False)
    return cp


def _gather_sum(table, idx_pad, fe, nf, n_coarse, bpw, n_pad):
    n_ch = bpw // _C
    assert n_ch % 2 == 0
    rows_a = (n_coarse // _NS) & ~7
    rows_b = n_coarse - _NS * rows_a
    mesh = plsc.VectorSubcoreMesh(core_axis_name="c", subcore_axis_name="s")

    @functools.partial(
        pl.kernel,
        out_type=jax.ShapeDtypeStruct((n_pad, nf), jnp.bfloat16),
        mesh=mesh,
        scratch_types=[
            pltpu.VMEM_SHARED((n_coarse, nf), jnp.bfloat16),
            pltpu.VMEM_SHARED((n_coarse, nf), jnp.bfloat16),
            pltpu.VMEM((bpw, nf), jnp.bfloat16),
            pltpu.VMEM((_C, nf), jnp.bfloat16),
            pltpu.VMEM((_C, nf), jnp.bfloat16),
            pltpu.VMEM((bpw,), jnp.int32),
            pltpu.VMEM((bpw,), jnp.int32),
            pltpu.SemaphoreType.DMA,
            pltpu.SemaphoreType.DMA,
            pltpu.SemaphoreType.DMA,
            pltpu.SemaphoreType.DMA,
            pltpu.SemaphoreType.DMA,
            pltpu.SemaphoreType.DMA,
        ],
        compiler_params=_sc_compiler_params(),
    )
    def body(table_hbm, idx_hbm, out_hbm,
             sh0, sh1, acc_v, r0, r1, ix0, ix1,
             ss0, ss1, sg0, sg1, si0, si1):
        cid = lax.axis_index("c")
        sid = lax.axis_index("s")
        wid = sid * _NC + cid
        sh = (sh0, sh1)
        rows = (r0, r1)
        ix = (ix0, ix1)
        ss = (ss0, ss1)
        sg = (sg0, sg1)
        si = (si0, si1)

        def stage_copies(k, q):
            a = sid * rows_a
            yield (table_hbm.at[k, pl.ds(a, rows_a)],
                   sh[q].at[pl.ds(a, rows_a)], ss[q])
            b = _NS * rows_a
            yield (table_hbm.at[k, pl.ds(b, rows_b)],
                   sh[q].at[pl.ds(b, rows_b)], ss[q])

        def stage_start(k, q):
            cps = list(stage_copies(k, q))
            pltpu.async_copy(*cps[0])

            @pl.when(sid == 0)
            def _():
                pltpu.async_copy(*cps[1])

        def stage_wait(k, q):
            cps = list(stage_copies(k, q))
            pltpu.make_async_copy(*cps[0]).wait()

            @pl.when(sid == 0)
            def _():
                pltpu.make_async_copy(*cps[1]).wait()

        def idx_start(k, kp):
            pltpu.async_copy(idx_hbm.at[k, pl.ds(wid * bpw, bpw)], ix[kp],
                             si[kp])

        def idx_wait(k, kp):
            pltpu.make_async_copy(idx_hbm.at[k, pl.ds(wid * bpw, bpw)],
                                  ix[kp], si[kp]).wait()

        def gather(k, q, c, p):
            src = sh[q].at[ix[k % 2].at[pl.ds(c * _C, _C)]]
            if k == 0:
                pltpu.async_copy(src, acc_v.at[pl.ds(c * _C, _C)], sg[p])
            else:
                pltpu.async_copy(src, rows[p], sg[p])

        def gather_wait(k, q, p):
            src = sh[q].at[ix[k % 2].at[pl.ds(0, _C)]]
            if k == 0:
                pltpu.make_async_copy(src, acc_v.at[pl.ds(0, _C)], sg[p]).wait()
            else:
                pltpu.make_async_copy(src, rows[p], sg[p]).wait()

        def accumulate(k, c, p):
            if k == 0:
                return

            @plsc.parallel_loop(0, _C, 2, unroll=4)
            def _row(r):
                for rr in range(2):
                    for j in range(nf // (2 * _LANES)):
                        w = 2 * _LANES
                        s = pl.ds(j * w, w)
                        row = c * _C + r + rr
                        acc_v[row, s] = acc_v[row, s] + rows[p][r + rr, s]

        stage_start(0, 0)
        stage_start(1, 1)
        idx_start(0, 0)
        idx_start(1, 1)
        idx_wait(0, 0)
        stage_wait(0, 0)
        plsc.subcore_barrier()

        for k in range(fe):
            q = k % 2
            gather(k, q, 0, 0)

            @pl.loop(0, n_ch, step=2)
            def _pair(c, k=k, q=q):
                gather_wait(k, q, 0)
                gather(k, q, c + 1, 1)
                accumulate(k, c, 0)
                gather_wait(k, q, 1)

                @pl.when(c + 2 < n_ch)
                def _():
                    gather(k, q, c + 2, 0)

                accumulate(k, c + 1, 1)

            if k + 2 <= fe - 1:
                idx_start(k + 2, q)
            if k + 1 <= fe - 1:
                idx_wait(k + 1, 1 - q)
                stage_wait(k + 1, 1 - q)
                plsc.subcore_barrier()
                if k + 2 <= fe - 1:
                    stage_start(k + 2, q)

        pltpu.sync_copy(acc_v, out_hbm.at[pl.ds(wid * bpw, bpw)])

    return body(table, idx_pad)


def kernel(lattice_coarse_values, neighbor_indices, weight):
    n_coarse, d = lattice_coarse_values.shape
    n_fine, fe = neighbor_indices.shape
    nf = weight.shape[1]

    vb = lattice_coarse_values.astype(jnp.bfloat16)
    w9b = weight.reshape(fe, d, nf).astype(jnp.bfloat16)
    table = _project_table(vb, w9b, m_block=2000)

    per_round = _NW * _C
    n_chunks = -(-n_fine // per_round)
    n_chunks += n_chunks % 2
    n_pad = n_chunks * per_round
    bpw = n_chunks * _C
    idx_pad = jnp.pad(neighbor_indices.T.astype(jnp.int32),
                      ((0, 0), (0, n_pad - n_fine)))

    out = _gather_sum(table, idx_pad, fe, nf, n_coarse, bpw, n_pad)
    return out[:n_fine].astype(jnp.float32)


# speedup vs baseline: 1.6173x; 1.1257x over previous
import dataclasses
import functools

import jax
import jax.numpy as jnp
from jax import lax
from jax.experimental import pallas as pl
from jax.experimental.pallas import tpu as pltpu
from jax.experimental.pallas import tpu_sc as plsc

_NC = 2
_NS = 16
_NW = _NC * _NS
_LANES = 16
_C = 64


def _project_table(values_bf, wp_bf, m_block):
    n_coarse, d = values_bf.shape
    n_out = wp_bf.shape[1]

    def body(v_ref, w_ref, o_ref):
        o_ref[...] = lax.dot_general(
            v_ref[...], w_ref[...], (((1,), (0,)), ((), ())),
            preferred_element_type=jnp.float32).astype(jnp.bfloat16)

    return pl.pallas_call(
        body,
        grid=(n_coarse // m_block,),
        in_specs=[
            pl.BlockSpec((m_block, d), lambda m: (m, 0)),
            pl.BlockSpec((d, n_out), lambda m: (0, 0)),
        ],
        out_specs=pl.BlockSpec((m_block, n_out), lambda m: (m, 0)),
        out_shape=jax.ShapeDtypeStruct((n_coarse, n_out), jnp.bfloat16),
    )(values_bf, wp_bf)


def _sc_compiler_params():
    cp = pltpu.CompilerParams(use_tc_tiling_on_sc=False)
    if "needs_layout_passes" in pltpu.CompilerParams.__dataclass_fields__:
        cp = dataclasses.replace(cp, needs_layout_passes=False)
    return cp


def _gather_sum(table, idx_pad, fe, nf, n_coarse, bpw, n_pad):
    n_ch = bpw // _C
    assert n_ch % 2 == 0
    rows_a = (n_coarse // _NS) & ~7
    rows_b = n_coarse - _NS * rows_a
    mesh = plsc.VectorSubcoreMesh(core_axis_name="c", subcore_axis_name="s")

    @functools.partial(
        pl.kernel,
        out_type=jax.ShapeDtypeStruct((n_pad, nf), jnp.bfloat16),
        mesh=mesh,
        scratch_types=[
            pltpu.VMEM_SHARED((n_coarse, nf), jnp.bfloat16),
            pltpu.VMEM_SHARED((n_coarse, nf), jnp.bfloat16),
            pltpu.VMEM((bpw, nf), jnp.bfloat16),
            pltpu.VMEM((_C, nf), jnp.bfloat16),
            pltpu.VMEM((_C, nf), jnp.bfloat16),
            pltpu.VMEM((bpw,), jnp.int32),
            pltpu.VMEM((bpw,), jnp.int32),
            pltpu.SemaphoreType.DMA,
            pltpu.SemaphoreType.DMA,
            pltpu.SemaphoreType.DMA,
            pltpu.SemaphoreType.DMA,
            pltpu.SemaphoreType.DMA,
            pltpu.SemaphoreType.DMA,
        ],
        compiler_params=_sc_compiler_params(),
    )
    def body(table_hbm, idx_hbm, out_hbm,
             sh0, sh1, acc_v, r0, r1, ix0, ix1,
             ss0, ss1, sg0, sg1, si0, si1):
        cid = lax.axis_index("c")
        sid = lax.axis_index("s")
        wid = sid * _NC + cid
        sh = (sh0, sh1)
        rows = (r0, r1)
        ix = (ix0, ix1)
        ss = (ss0, ss1)
        sg = (sg0, sg1)
        si = (si0, si1)

        def stage_copies(k, q):
            a = sid * rows_a
            cols = pl.ds(k * nf, nf)
            yield (table_hbm.at[pl.ds(a, rows_a), cols],
                   sh[q].at[pl.ds(a, rows_a)], ss[q])
            b = _NS * rows_a
            yield (table_hbm.at[pl.ds(b, rows_b), cols],
                   sh[q].at[pl.ds(b, rows_b)], ss[q])

        def stage_start(k, q):
            cps = list(stage_copies(k, q))
            pltpu.async_copy(*cps[0])

            @pl.when(sid == 0)
            def _():
                pltpu.async_copy(*cps[1])

        def stage_wait(k, q):
            cps = list(stage_copies(k, q))
            pltpu.make_async_copy(*cps[0]).wait()

            @pl.when(sid == 0)
            def _():
                pltpu.make_async_copy(*cps[1]).wait()

        def idx_start(k, kp):
            pltpu.async_copy(idx_hbm.at[k, pl.ds(wid * bpw, bpw)], ix[kp],
                             si[kp])

        def idx_wait(k, kp):
            pltpu.make_async_copy(idx_hbm.at[k, pl.ds(wid * bpw, bpw)],
                                  ix[kp], si[kp]).wait()

        def gather(k, q, c, p):
            src = sh[q].at[ix[k % 2].at[pl.ds(c * _C, _C)]]
            if k == 0:
                pltpu.async_copy(src, acc_v.at[pl.ds(c * _C, _C)], sg[p])
            else:
                pltpu.async_copy(src, rows[p], sg[p])

        def gather_wait(k, q, p):
            src = sh[q].at[ix[k % 2].at[pl.ds(0, _C)]]
            if k == 0:
                pltpu.make_async_copy(src, acc_v.at[pl.ds(0, _C)], sg[p]).wait()
            else:
                pltpu.make_async_copy(src, rows[p], sg[p]).wait()

        def accumulate(k, c, p):
            if k == 0:
                return

            @plsc.parallel_loop(0, _C, 2, unroll=4)
            def _row(r):
                for rr in range(2):
                    for j in range(nf // (2 * _LANES)):
                        w = 2 * _LANES
                        s = pl.ds(j * w, w)
                        row = c * _C + r + rr
                        acc_v[row, s] = acc_v[row, s] + rows[p][r + rr, s]

        stage_start(0, 0)
        stage_start(1, 1)
        idx_start(0, 0)
        idx_start(1, 1)
        idx_wait(0, 0)
        stage_wait(0, 0)
        plsc.subcore_barrier()

        for k in range(fe):
            q = k % 2
            gather(k, q, 0, 0)

            @pl.loop(0, n_ch, step=2)
            def _pair(c, k=k, q=q):
                gather_wait(k, q, 0)
                gather(k, q, c + 1, 1)
                accumulate(k, c, 0)
                gather_wait(k, q, 1)

                @pl.when(c + 2 < n_ch)
                def _():
                    gather(k, q, c + 2, 0)

                accumulate(k, c + 1, 1)

            if k + 2 <= fe - 1:
                idx_start(k + 2, q)
            if k + 1 <= fe - 1:
                idx_wait(k + 1, 1 - q)
                stage_wait(k + 1, 1 - q)
                plsc.subcore_barrier()
                if k + 2 <= fe - 1:
                    stage_start(k + 2, q)

        pltpu.sync_copy(acc_v, out_hbm.at[pl.ds(wid * bpw, bpw)])

    return body(table, idx_pad)


def kernel(lattice_coarse_values, neighbor_indices, weight):
    n_coarse, d = lattice_coarse_values.shape
    n_fine, fe = neighbor_indices.shape
    nf = weight.shape[1]

    vb = lattice_coarse_values.astype(jnp.bfloat16)
    wp = weight.reshape(fe, d, nf).transpose(1, 0, 2).reshape(d, fe * nf)
    table = _project_table(vb, wp.astype(jnp.bfloat16), m_block=2000)

    per_round = _NW * _C
    n_chunks = -(-n_fine // per_round)
    n_chunks += n_chunks % 2
    n_pad = n_chunks * per_round
    bpw = n_chunks * _C
    idx_pad = jnp.pad(neighbor_indices.T.astype(jnp.int32),
                      ((0, 0), (0, n_pad - n_fine)))

    out = _gather_sum(table, idx_pad, fe, nf, n_coarse, bpw, n_pad)
    return out[:n_fine].astype(jnp.float32)


# speedup vs baseline: 1.7993x; 1.1125x over previous
import dataclasses
import functools

import jax
import jax.numpy as jnp
from jax import lax
from jax.experimental import pallas as pl
from jax.experimental.pallas import tpu as pltpu
from jax.experimental.pallas import tpu_sc as plsc

_NC = 2
_NS = 16
_NW = _NC * _NS
_LANES = 16
_C = 64


def _project_table(values_bf, wp_bf, m_block):
    n_coarse, d = values_bf.shape
    n_out = wp_bf.shape[1]

    def body(v_ref, w_ref, o_ref):
        o_ref[...] = lax.dot_general(
            v_ref[...], w_ref[...], (((1,), (0,)), ((), ())),
            preferred_element_type=jnp.float32).astype(jnp.bfloat16)

    return pl.pallas_call(
        body,
        grid=(n_coarse // m_block,),
        in_specs=[
            pl.BlockSpec((m_block, d), lambda m: (m, 0)),
            pl.BlockSpec((d, n_out), lambda m: (0, 0)),
        ],
        out_specs=pl.BlockSpec((m_block, n_out), lambda m: (m, 0)),
        out_shape=jax.ShapeDtypeStruct((n_coarse, n_out), jnp.bfloat16),
    )(values_bf, wp_bf)


def _sc_compiler_params():
    cp = pltpu.CompilerParams(use_tc_tiling_on_sc=False)
    if "needs_layout_passes" in pltpu.CompilerParams.__dataclass_fields__:
        cp = dataclasses.replace(cp, needs_layout_passes=False)
    return cp


def _gather_sum(table, idx_pad, fe, nf, n_coarse, bpw, n_pad):
    n_ch = bpw // _C
    assert n_ch % 2 == 0
    rows_a = (n_coarse // _NS) & ~7
    rows_b = n_coarse - _NS * rows_a
    mesh = plsc.VectorSubcoreMesh(core_axis_name="c", subcore_axis_name="s")

    @functools.partial(
        pl.kernel,
        out_type=jax.ShapeDtypeStruct((n_pad * nf,), jnp.float32),
        mesh=mesh,
        scratch_types=[
            pltpu.VMEM_SHARED((n_coarse, nf), jnp.bfloat16),
            pltpu.VMEM_SHARED((n_coarse, nf), jnp.bfloat16),
            pltpu.VMEM((bpw, nf), jnp.bfloat16),
            pltpu.VMEM((_C, nf), jnp.bfloat16),
            pltpu.VMEM((_C, nf), jnp.bfloat16),
            pltpu.VMEM((bpw,), jnp.int32),
            pltpu.VMEM((bpw,), jnp.int32),
            pltpu.VMEM((_C * nf,), jnp.float32),
            pltpu.VMEM((_C * nf,), jnp.float32),
            pltpu.SemaphoreType.DMA,
            pltpu.SemaphoreType.DMA,
            pltpu.SemaphoreType.DMA,
            pltpu.SemaphoreType.DMA,
            pltpu.SemaphoreType.DMA,
            pltpu.SemaphoreType.DMA,
            pltpu.SemaphoreType.DMA,
            pltpu.SemaphoreType.DMA,
        ],
        compiler_params=_sc_compiler_params(),
    )
    def body(table_hbm, idx_hbm, out_hbm,
             sh0, sh1, acc_v, r0, r1, ix0, ix1, st0, st1,
             ss0, ss1, sg0, sg1, si0, si1, so0, so1):
        cid = lax.axis_index("c")
        sid = lax.axis_index("s")
        wid = sid * _NC + cid
        sh = (sh0, sh1)
        rows = (r0, r1)
        ix = (ix0, ix1)
        stg = (st0, st1)
        ss = (ss0, ss1)
        sg = (sg0, sg1)
        si = (si0, si1)
        so = (so0, so1)

        def stage_copies(k, q):
            a = sid * rows_a
            cols = pl.ds(k * nf, nf)
            yield (table_hbm.at[pl.ds(a, rows_a), cols],
                   sh[q].at[pl.ds(a, rows_a)], ss[q])
            b = _NS * rows_a
            yield (table_hbm.at[pl.ds(b, rows_b), cols],
                   sh[q].at[pl.ds(b, rows_b)], ss[q])

        def stage_start(k, q):
            cps = list(stage_copies(k, q))
            pltpu.async_copy(*cps[0])

            @pl.when(sid == 0)
            def _():
                pltpu.async_copy(*cps[1])

        def stage_wait(k, q):
            cps = list(stage_copies(k, q))
            pltpu.make_async_copy(*cps[0]).wait()

            @pl.when(sid == 0)
            def _():
                pltpu.make_async_copy(*cps[1]).wait()

        def idx_start(k, kp):
            pltpu.async_copy(idx_hbm.at[k, pl.ds(wid * bpw, bpw)], ix[kp],
                             si[kp])

        def idx_wait(k, kp):
            pltpu.make_async_copy(idx_hbm.at[k, pl.ds(wid * bpw, bpw)],
                                  ix[kp], si[kp]).wait()

        def gather(k, q, c, p):
            src = sh[q].at[ix[k % 2].at[pl.ds(c * _C, _C)]]
            if k == 0:
                pltpu.async_copy(src, acc_v.at[pl.ds(c * _C, _C)], sg[p])
            else:
                pltpu.async_copy(src, rows[p], sg[p])

        def gather_wait(k, q, p):
            src = sh[q].at[ix[k % 2].at[pl.ds(0, _C)]]
            if k == 0:
                pltpu.make_async_copy(src, acc_v.at[pl.ds(0, _C)], sg[p]).wait()
            else:
                pltpu.make_async_copy(src, rows[p], sg[p]).wait()

        def accumulate(k, c, p):
            if k == 0:
                return
            w = 2 * _LANES

            if k == fe - 1:
                @pl.when(c >= 2)
                def _():
                    pltpu.make_async_copy(
                        stg[p], out_hbm.at[pl.ds(0, _C * nf)], so[p]).wait()

                @plsc.parallel_loop(0, _C, 1, unroll=4)
                def _row(r):
                    for j in range(nf // w):
                        s = pl.ds(j * w, w)
                        v = acc_v[c * _C + r, s] + rows[p][r, s]
                        a, b = plsc.unpack(v, format=plsc.PackFormat.INTERLEAVED)
                        o = r * nf + j * w
                        stg[p][pl.ds(o, _LANES)] = a
                        stg[p][pl.ds(o + _LANES, _LANES)] = b

                pltpu.async_copy(
                    stg[p],
                    out_hbm.at[pl.ds((wid * bpw + c * _C) * nf, _C * nf)],
                    so[p])
                return

            @plsc.parallel_loop(0, _C, 2, unroll=4)
            def _row(r):
                for rr in range(2):
                    for j in range(nf // w):
                        s = pl.ds(j * w, w)
                        row = c * _C + r + rr
                        acc_v[row, s] = acc_v[row, s] + rows[p][r + rr, s]

        stage_start(0, 0)
        stage_start(1, 1)
        idx_start(0, 0)
        idx_start(1, 1)
        idx_wait(0, 0)
        stage_wait(0, 0)
        plsc.subcore_barrier()

        for k in range(fe):
            q = k % 2
            gather(k, q, 0, 0)

            @pl.loop(0, n_ch, step=2)
            def _pair(c, k=k, q=q):
                gather_wait(k, q, 0)
                gather(k, q, c + 1, 1)
                accumulate(k, c, 0)
                gather_wait(k, q, 1)

                @pl.when(c + 2 < n_ch)
                def _():
                    gather(k, q, c + 2, 0)

                accumulate(k, c + 1, 1)

            if k + 2 <= fe - 1:
                idx_start(k + 2, q)
            if k + 1 <= fe - 1:
                idx_wait(k + 1, 1 - q)
                stage_wait(k + 1, 1 - q)
                plsc.subcore_barrier()
                if k + 2 <= fe - 1:
                    stage_start(k + 2, q)

        for p in range(2):
            pltpu.make_async_copy(
                stg[p], out_hbm.at[pl.ds(0, _C * nf)], so[p]).wait()

    return body(table, idx_pad)


def kernel(lattice_coarse_values, neighbor_indices, weight):
    n_coarse, d = lattice_coarse_values.shape
    n_fine, fe = neighbor_indices.shape
    nf = weight.shape[1]

    vb = lattice_coarse_values.astype(jnp.bfloat16)
    wp = weight.reshape(fe, d, nf).transpose(1, 0, 2).reshape(d, fe * nf)
    t16 = jnp.arange(16, dtype=jnp.int32)
    perm32 = jnp.stack([t16, t16 + 16], axis=1).reshape(32)
    permnf = jnp.concatenate([perm32 + 32 * h for h in range(nf // 32)])
    permfull = (jnp.arange(fe, dtype=jnp.int32)[:, None] * nf
                + permnf[None, :]).reshape(-1)
    wp = wp[:, permfull]
    table = _project_table(vb, wp.astype(jnp.bfloat16), m_block=2000)

    per_round = _NW * _C
    n_chunks = -(-n_fine // per_round)
    n_chunks += n_chunks % 2
    n_pad = n_chunks * per_round
    bpw = n_chunks * _C
    idx_pad = jnp.pad(neighbor_indices.T.astype(jnp.int32),
                      ((0, 0), (0, n_pad - n_fine)))

    out = _gather_sum(table, idx_pad, fe, nf, n_coarse, bpw, n_pad)
    return out[:n_fine * nf].reshape(n_fine, nf)


# speedup vs baseline: 1.8030x; 1.0021x over previous
import dataclasses
import functools

import jax
import jax.numpy as jnp
from jax import lax
from jax.experimental import pallas as pl
from jax.experimental.pallas import tpu as pltpu
from jax.experimental.pallas import tpu_sc as plsc

_NC = 2
_NS = 16
_NW = _NC * _NS
_LANES = 16
_C = 64


def _project_table(values_bf, wp_bf, m_block):
    n_coarse, d = values_bf.shape
    n_out = wp_bf.shape[1]

    def body(v_ref, w_ref, o_ref):
        o_ref[...] = lax.dot_general(
            v_ref[...].astype(jnp.bfloat16), w_ref[...],
            (((1,), (0,)), ((), ())),
            preferred_element_type=jnp.float32).astype(jnp.bfloat16)

    return pl.pallas_call(
        body,
        grid=(n_coarse // m_block,),
        in_specs=[
            pl.BlockSpec((m_block, d), lambda m: (m, 0)),
            pl.BlockSpec((d, n_out), lambda m: (0, 0)),
        ],
        out_specs=pl.BlockSpec((m_block, n_out), lambda m: (m, 0)),
        out_shape=jax.ShapeDtypeStruct((n_coarse, n_out), jnp.bfloat16),
    )(values_bf, wp_bf)


def _sc_compiler_params():
    cp = pltpu.CompilerParams(use_tc_tiling_on_sc=False)
    if "needs_layout_passes" in pltpu.CompilerParams.__dataclass_fields__:
        cp = dataclasses.replace(cp, needs_layout_passes=False)
    return cp


def _gather_sum(table, idx_t, fe, nf, n_coarse, bpw, n_pad, n_fine):
    n_ch = bpw // _C
    assert n_ch % 2 == 0
    full_tiles = n_fine // bpw
    rem = n_fine - full_tiles * bpw
    assert rem % _LANES == 0 and full_tiles < _NW
    rows_a = (n_coarse // _NS) & ~7
    rows_b = n_coarse - _NS * rows_a
    mesh = plsc.VectorSubcoreMesh(core_axis_name="c", subcore_axis_name="s")

    @functools.partial(
        pl.kernel,
        out_type=jax.ShapeDtypeStruct((n_pad * nf,), jnp.float32),
        mesh=mesh,
        scratch_types=[
            pltpu.VMEM_SHARED((n_coarse, nf), jnp.bfloat16),
            pltpu.VMEM_SHARED((n_coarse, nf), jnp.bfloat16),
            pltpu.VMEM((bpw, nf), jnp.bfloat16),
            pltpu.VMEM((_C, nf), jnp.bfloat16),
            pltpu.VMEM((_C, nf), jnp.bfloat16),
            pltpu.VMEM((bpw,), jnp.int32),
            pltpu.VMEM((bpw,), jnp.int32),
            pltpu.VMEM((_C * nf,), jnp.float32),
            pltpu.VMEM((_C * nf,), jnp.float32),
            pltpu.SemaphoreType.DMA,
            pltpu.SemaphoreType.DMA,
            pltpu.SemaphoreType.DMA,
            pltpu.SemaphoreType.DMA,
            pltpu.SemaphoreType.DMA,
            pltpu.SemaphoreType.DMA,
            pltpu.SemaphoreType.DMA,
            pltpu.SemaphoreType.DMA,
        ],
        compiler_params=_sc_compiler_params(),
    )
    def body(table_hbm, idx_hbm, out_hbm,
             sh0, sh1, acc_v, r0, r1, ix0, ix1, st0, st1,
             ss0, ss1, sg0, sg1, si0, si1, so0, so1):
        cid = lax.axis_index("c")
        sid = lax.axis_index("s")
        wid = sid * _NC + cid
        sh = (sh0, sh1)
        rows = (r0, r1)
        ix = (ix0, ix1)
        stg = (st0, st1)
        ss = (ss0, ss1)
        sg = (sg0, sg1)
        si = (si0, si1)
        so = (so0, so1)

        def stage_copies(k, q):
            a = sid * rows_a
            cols = pl.ds(k * nf, nf)
            yield (table_hbm.at[pl.ds(a, rows_a), cols],
                   sh[q].at[pl.ds(a, rows_a)], ss[q])
            b = _NS * rows_a
            yield (table_hbm.at[pl.ds(b, rows_b), cols],
                   sh[q].at[pl.ds(b, rows_b)], ss[q])

        def stage_start(k, q):
            cps = list(stage_copies(k, q))
            pltpu.async_copy(*cps[0])

            @pl.when(sid == 0)
            def _():
                pltpu.async_copy(*cps[1])

        def stage_wait(k, q):
            cps = list(stage_copies(k, q))
            pltpu.make_async_copy(*cps[0]).wait()

            @pl.when(sid == 0)
            def _():
                pltpu.make_async_copy(*cps[1]).wait()

        zeros16 = jnp.zeros((_LANES,), jnp.int32)

        def idx_start(k, kp):
            @pl.when(wid < full_tiles)
            def _():
                pltpu.async_copy(idx_hbm.at[k, pl.ds(wid * bpw, bpw)],
                                 ix[kp], si[kp])

            @pl.when(wid == full_tiles)
            def _():
                pltpu.async_copy(
                    idx_hbm.at[k, pl.ds(full_tiles * bpw, rem)],
                    ix[kp].at[pl.ds(0, rem)], si[kp])

                @plsc.parallel_loop(rem // _LANES, bpw // _LANES, 1, unroll=4)
                def _fill(t):
                    ix[kp][pl.ds(t * _LANES, _LANES)] = zeros16

            @pl.when(wid > full_tiles)
            def _():
                @plsc.parallel_loop(0, bpw // _LANES, 1, unroll=4)
                def _fill(t):
                    ix[kp][pl.ds(t * _LANES, _LANES)] = zeros16

        def idx_wait(k, kp):
            @pl.when(wid < full_tiles)
            def _():
                pltpu.make_async_copy(idx_hbm.at[k, pl.ds(wid * bpw, bpw)],
                                      ix[kp], si[kp]).wait()

            @pl.when(wid == full_tiles)
            def _():
                pltpu.make_async_copy(
                    idx_hbm.at[k, pl.ds(full_tiles * bpw, rem)],
                    ix[kp].at[pl.ds(0, rem)], si[kp]).wait()

        def gather(k, q, c, p):
            src = sh[q].at[ix[k % 2].at[pl.ds(c * _C, _C)]]
            if k == 0:
                pltpu.async_copy(src, acc_v.at[pl.ds(c * _C, _C)], sg[p])
            else:
                pltpu.async_copy(src, rows[p], sg[p])

        def gather_wait(k, q, p):
            src = sh[q].at[ix[k % 2].at[pl.ds(0, _C)]]
            if k == 0:
                pltpu.make_async_copy(src, acc_v.at[pl.ds(0, _C)], sg[p]).wait()
            else:
                pltpu.make_async_copy(src, rows[p], sg[p]).wait()

        def accumulate(k, c, p):
            if k == 0:
                return
            w = 2 * _LANES

            if k == fe - 1:
                @pl.when(c >= 2)
                def _():
                    pltpu.make_async_copy(
                        stg[p], out_hbm.at[pl.ds(0, _C * nf)], so[p]).wait()

                @plsc.parallel_loop(0, _C, 1, unroll=4)
                def _row(r):
                    for j in range(nf // w):
                        s = pl.ds(j * w, w)
                        v = acc_v[c * _C + r, s] + rows[p][r, s]
                        a, b = plsc.unpack(v, format=plsc.PackFormat.INTERLEAVED)
                        o = r * nf + j * w
                        stg[p][pl.ds(o, _LANES)] = a
                        stg[p][pl.ds(o + _LANES, _LANES)] = b

                pltpu.async_copy(
                    stg[p],
                    out_hbm.at[pl.ds((wid * bpw + c * _C) * nf, _C * nf)],
                    so[p])
                return

            @plsc.parallel_loop(0, _C, 2, unroll=4)
            def _row(r):
                for rr in range(2):
                    for j in range(nf // w):
                        s = pl.ds(j * w, w)
                        row = c * _C + r + rr
                        acc_v[row, s] = acc_v[row, s] + rows[p][r + rr, s]

        stage_start(0, 0)
        stage_start(1, 1)
        idx_start(0, 0)
        idx_start(1, 1)
        idx_wait(0, 0)
        stage_wait(0, 0)
        plsc.subcore_barrier()

        for k in range(fe):
            q = k % 2
            gather(k, q, 0, 0)

            @pl.loop(0, n_ch, step=2)
            def _pair(c, k=k, q=q):
                gather_wait(k, q, 0)
                gather(k, q, c + 1, 1)
                accumulate(k, c, 0)
                gather_wait(k, q, 1)

                @pl.when(c + 2 < n_ch)
                def _():
                    gather(k, q, c + 2, 0)

                accumulate(k, c + 1, 1)

            if k + 2 <= fe - 1:
                idx_start(k + 2, q)
            if k + 1 <= fe - 1:
                idx_wait(k + 1, 1 - q)
                stage_wait(k + 1, 1 - q)
                plsc.subcore_barrier()
                if k + 2 <= fe - 1:
                    stage_start(k + 2, q)

        for p in range(2):
            pltpu.make_async_copy(
                stg[p], out_hbm.at[pl.ds(0, _C * nf)], so[p]).wait()

    return body(table, idx_t)


def kernel(lattice_coarse_values, neighbor_indices, weight):
    n_coarse, d = lattice_coarse_values.shape
    n_fine, fe = neighbor_indices.shape
    nf = weight.shape[1]

    wp = weight.reshape(fe, d, nf).transpose(1, 0, 2).reshape(d, fe * nf)
    t16 = jnp.arange(16, dtype=jnp.int32)
    perm32 = jnp.stack([t16, t16 + 16], axis=1).reshape(32)
    permnf = jnp.concatenate([perm32 + 32 * h for h in range(nf // 32)])
    permfull = (jnp.arange(fe, dtype=jnp.int32)[:, None] * nf
                + permnf[None, :]).reshape(-1)
    wp = wp[:, permfull]
    table = _project_table(lattice_coarse_values, wp.astype(jnp.bfloat16),
                           m_block=2000)

    per_round = _NW * _C
    n_chunks = -(-n_fine // per_round)
    n_chunks += n_chunks % 2
    n_pad = n_chunks * per_round
    bpw = n_chunks * _C
    idx_t = neighbor_indices.T.astype(jnp.int32)

    out = _gather_sum(table, idx_t, fe, nf, n_coarse, bpw, n_pad, n_fine)
    return out[:n_fine * nf].reshape(n_fine, nf)
